# baseline (speedup 1.0000x reference)
import jax
import jax.numpy as jnp
from jax.experimental import pallas as pl
from jax.experimental.pallas import tpu as pltpu

_TOPK = 16
_WS = 4
_WIN = 64
_NS = 1024
_C = 384
_G4 = 96
_NTOK = _TOPK * _WS * _WS
_NB = 4


def _body(z_ref, x_ref, pool_ref, eye_ref, wdT_ref, bd_ref, wuT_ref, bu_ref,
          out_ref, xg_ref):
    for b in range(_NB):
        z = z_ref[b]
        x = x_ref[b]

        zmax = jnp.max(z, axis=0, keepdims=True)
        zr = zmax.astype(jnp.bfloat16).astype(jnp.float32)
        xb = x.astype(jnp.bfloat16)
        A = jnp.dot(pool_ref[...], xb,
                    preferred_element_type=jnp.float32)
        wsum = jnp.sum(A * zr, axis=1, keepdims=True)
        wrow = jax.lax.dot_general(wsum, eye_ref[...],
                                   dimension_numbers=(((0,), (0,)), ((), ())),
                                   preferred_element_type=jnp.float32,
                                   precision=jax.lax.Precision.HIGHEST)

        iv = jax.lax.broadcasted_iota(jnp.int32, (_WIN, _WIN), 0)
        iw = jax.lax.broadcasted_iota(jnp.int32, (_WIN, _WIN), 1)
        beats = (wsum > wrow) | ((wsum == wrow) & (iv < iw))
        rank = jnp.sum(beats.astype(jnp.int32), axis=0, keepdims=True)

        iota_row = jax.lax.broadcasted_iota(jnp.int32, (1, _WIN), 1)
        for i in range(_TOPK):
            sel = jnp.max(jnp.where(rank == i, iota_row, 0))
            wh = sel // 8
            ww = sel % 8
            aligned = wh * 128 + (ww // 2) * 8
            odd = (ww % 2) == 1
            for r in range(_WS):
                chunk = x_ref[b, pl.ds(aligned + r * 32, 8), :]
                xg_ref[pl.ds(b * _NTOK + i * 16 + r * 4, 4), :] = jnp.where(
                    odd, chunk[4:8], chunk[0:4])

        x_g = xg_ref[pl.ds(b * _NTOK, _NTOK), :]

        t = jnp.dot(x_g.astype(jnp.bfloat16), wdT_ref[...],
                    preferred_element_type=jnp.float32)
        t = t + bd_ref[...]

        zero1 = jnp.zeros((1, _G4), jnp.float32)
        zero4 = jnp.zeros((4, _G4), jnp.float32)
        tp1 = jnp.concatenate([t[1:], zero1], axis=0)
        tm1 = jnp.concatenate([zero1, t[:-1]], axis=0)
        tp4 = jnp.concatenate([t[4:], zero4], axis=0)
        tm4 = jnp.concatenate([zero4, t[:-4]], axis=0)
        j = jax.lax.broadcasted_iota(jnp.int32, (_NTOK, 1), 0)
        cpos = j % 4
        rpos = (j % 16) // 4
        s0 = jnp.where(cpos < 3, tp1, 0.0)
        s1 = jnp.where(cpos > 0, tm1, 0.0)
        s2 = jnp.where(rpos < 3, tp4, 0.0)
        s3 = jnp.where(rpos > 0, tm4, 0.0)
        ch = jax.lax.broadcasted_iota(jnp.int32, (_NTOK, _G4), 1)
        s = jnp.where(ch < 24, s0,
                      jnp.where(ch < 48, s1, jnp.where(ch < 72, s2, s3)))

        su = jnp.dot(s.astype(jnp.bfloat16), wuT_ref[...],
                     preferred_element_type=jnp.float32)
        out_ref[b] = x_g + su + bu_ref[...]


def kernel(z, x, w_down, b_down, w_up, b_up):
    B, N_t, C = z.shape
    N_s = x.shape[1]

    n = jax.lax.broadcasted_iota(jnp.int32, (_WIN, _NS), 1)
    w = jax.lax.broadcasted_iota(jnp.int32, (_WIN, _NS), 0)
    pool = (((n // 128) == (w // 8)) & (((n % 32) // 4) == (w % 8))
            ).astype(jnp.bfloat16)
    eye = jnp.eye(_WIN, dtype=jnp.float32)

    out = pl.pallas_call(
        _body,
        grid=(B // _NB,),
        in_specs=[
            pl.BlockSpec((_NB, N_t, C), lambda b: (b, 0, 0)),
            pl.BlockSpec((_NB, N_s, C), lambda b: (b, 0, 0)),
            pl.BlockSpec((_WIN, _NS), lambda b: (0, 0)),
            pl.BlockSpec((_WIN, _WIN), lambda b: (0, 0)),
            pl.BlockSpec((C, _G4), lambda b: (0, 0)),
            pl.BlockSpec((1, _G4), lambda b: (0, 0)),
            pl.BlockSpec((_G4, C), lambda b: (0, 0)),
            pl.BlockSpec((1, C), lambda b: (0, 0)),
        ],
        out_specs=pl.BlockSpec((_NB, _NTOK, C), lambda b: (b, 0, 0)),
        out_shape=jax.ShapeDtypeStruct((B, _NTOK, C), jnp.float32),
        scratch_shapes=[pltpu.VMEM((_NB * _NTOK, C), jnp.float32)],
        compiler_params=pltpu.CompilerParams(
            dimension_semantics=("arbitrary",),
        ),
    )(z, x, pool, eye, w_down.T.astype(jnp.bfloat16), b_down.reshape(1, -1),
      w_up.T.astype(jnp.bfloat16), b_up.reshape(1, -1))
    return out


# speedup vs baseline: 1.1319x; 1.1319x over previous
import jax
import jax.numpy as jnp
from jax.experimental import pallas as pl
from jax.experimental.pallas import tpu as pltpu

_TOPK = 16
_WS = 4
_WIN = 64
_NS = 1024
_C = 384
_G4 = 96
_NTOK = _TOPK * _WS * _WS
_NB = 4


def _body(z_ref, x_ref, pool_ref, eye_ref, wdT_ref, bd_ref, wuT_ref, bu_ref,
          out_ref, xg_ref):
    for b in range(_NB):
        z = z_ref[b]
        x = x_ref[b]

        zmax = jnp.max(z, axis=0, keepdims=True)
        zr = zmax.astype(jnp.bfloat16).astype(jnp.float32)
        xb = x.astype(jnp.bfloat16)
        A = jnp.dot(pool_ref[...], xb,
                    preferred_element_type=jnp.float32)
        wsum = jnp.sum(A * zr, axis=1, keepdims=True)
        wrow = jnp.transpose(wsum)

        iv = jax.lax.broadcasted_iota(jnp.int32, (_WIN, _WIN), 0)
        iw = jax.lax.broadcasted_iota(jnp.int32, (_WIN, _WIN), 1)
        beats = (wsum > wrow) | ((wsum == wrow) & (iv < iw))
        rank = jnp.sum(beats.astype(jnp.int32), axis=0, keepdims=True)

        iota_row = jax.lax.broadcasted_iota(jnp.int32, (1, _WIN), 1)
        for i in range(_TOPK):
            sel = jnp.max(jnp.where(rank == i, iota_row, 0))
            wh = sel // 8
            ww = sel % 8
            aligned = wh * 128 + (ww // 2) * 8
            odd = (ww % 2) == 1
            for r in range(_WS):
                chunk = x_ref[b, pl.ds(aligned + r * 32, 8), :]
                xg_ref[pl.ds(b * _NTOK + i * 16 + r * 4, 4), :] = jnp.where(
                    odd, chunk[4:8], chunk[0:4])

        x_g = xg_ref[pl.ds(b * _NTOK, _NTOK), :]

        t = jnp.dot(x_g.astype(jnp.bfloat16), wdT_ref[...],
                    preferred_element_type=jnp.float32)
        t = t + bd_ref[...]

        zero1 = jnp.zeros((1, _G4), jnp.float32)
        zero4 = jnp.zeros((4, _G4), jnp.float32)
        tp1 = jnp.concatenate([t[1:], zero1], axis=0)
        tm1 = jnp.concatenate([zero1, t[:-1]], axis=0)
        tp4 = jnp.concatenate([t[4:], zero4], axis=0)
        tm4 = jnp.concatenate([zero4, t[:-4]], axis=0)
        j = jax.lax.broadcasted_iota(jnp.int32, (_NTOK, 1), 0)
        cpos = j % 4
        rpos = (j % 16) // 4
        s0 = jnp.where(cpos < 3, tp1, 0.0)
        s1 = jnp.where(cpos > 0, tm1, 0.0)
        s2 = jnp.where(rpos < 3, tp4, 0.0)
        s3 = jnp.where(rpos > 0, tm4, 0.0)
        ch = jax.lax.broadcasted_iota(jnp.int32, (_NTOK, _G4), 1)
        s = jnp.where(ch < 24, s0,
                      jnp.where(ch < 48, s1, jnp.where(ch < 72, s2, s3)))

        su = jnp.dot(s.astype(jnp.bfloat16), wuT_ref[...],
                     preferred_element_type=jnp.float32)
        out_ref[b] = x_g + su + bu_ref[...]


def kernel(z, x, w_down, b_down, w_up, b_up):
    B, N_t, C = z.shape
    N_s = x.shape[1]

    n = jax.lax.broadcasted_iota(jnp.int32, (_WIN, _NS), 1)
    w = jax.lax.broadcasted_iota(jnp.int32, (_WIN, _NS), 0)
    pool = (((n // 128) == (w // 8)) & (((n % 32) // 4) == (w % 8))
            ).astype(jnp.bfloat16)
    eye = jnp.eye(_WIN, dtype=jnp.float32)

    out = pl.pallas_call(
        _body,
        grid=(B // _NB,),
        in_specs=[
            pl.BlockSpec((_NB, N_t, C), lambda b: (b, 0, 0)),
            pl.BlockSpec((_NB, N_s, C), lambda b: (b, 0, 0)),
            pl.BlockSpec((_WIN, _NS), lambda b: (0, 0)),
            pl.BlockSpec((_WIN, _WIN), lambda b: (0, 0)),
            pl.BlockSpec((C, _G4), lambda b: (0, 0)),
            pl.BlockSpec((1, _G4), lambda b: (0, 0)),
            pl.BlockSpec((_G4, C), lambda b: (0, 0)),
            pl.BlockSpec((1, C), lambda b: (0, 0)),
        ],
        out_specs=pl.BlockSpec((_NB, _NTOK, C), lambda b: (b, 0, 0)),
        out_shape=jax.ShapeDtypeStruct((B, _NTOK, C), jnp.float32),
        scratch_shapes=[pltpu.VMEM((_NB * _NTOK, C), jnp.float32)],
        compiler_params=pltpu.CompilerParams(
            dimension_semantics=("arbitrary",),
        ),
    )(z, x, pool, eye, w_down.T.astype(jnp.bfloat16), b_down.reshape(1, -1),
      w_up.T.astype(jnp.bfloat16), b_up.reshape(1, -1))
    return out


# speedup vs baseline: 1.4501x; 1.2811x over previous
import jax
import jax.numpy as jnp
from jax.experimental import pallas as pl
from jax.experimental.pallas import tpu as pltpu

_TOPK = 16
_WS = 4
_WIN = 64
_NS = 1024
_C = 384
_G4 = 96
_NTOK = _TOPK * _WS * _WS
_NB = 4


def _body(z_ref, x_ref, pool_ref, eye_ref, wdT_ref, bd_ref, wuT_ref, bu_ref,
          out_ref, xg_ref):

    ranks = []
    for b in range(_NB):
        zmax = jnp.max(z_ref[b], axis=0, keepdims=True)
        zr = zmax.astype(jnp.bfloat16).astype(jnp.float32)
        xb = x_ref[b].astype(jnp.bfloat16)
        A = jnp.dot(pool_ref[...], xb,
                    preferred_element_type=jnp.float32)
        wsum = jnp.sum(A * zr, axis=1, keepdims=True)
        wrow = jnp.transpose(wsum)

        iv = jax.lax.broadcasted_iota(jnp.int32, (_WIN, _WIN), 0)
        iw = jax.lax.broadcasted_iota(jnp.int32, (_WIN, _WIN), 1)
        beats = (wsum > wrow) | ((wsum == wrow) & (iv < iw))
        ranks.append(jnp.sum(beats.astype(jnp.int32), axis=0,
                             keepdims=True))

    iota_row = jax.lax.broadcasted_iota(jnp.int32, (1, _WIN), 1)
    for i in range(_TOPK):
        for b in range(_NB):
            sel = jnp.max(jnp.where(ranks[b] == i, iota_row, 0))
            wh = sel // 8
            ww = sel % 8
            aligned = wh * 128 + (ww // 2) * 8
            odd = (ww % 2) == 1
            for r in range(_WS):
                chunk = x_ref[b, pl.ds(aligned + r * 32, 8), :]
                xg_ref[pl.ds(b * _NTOK + i * 16 + r * 4, 4), :] = jnp.where(
                    odd, chunk[4:8], chunk[0:4])

    for b in range(_NB):
        x_g = xg_ref[pl.ds(b * _NTOK, _NTOK), :]

        t = jnp.dot(x_g.astype(jnp.bfloat16), wdT_ref[...],
                    preferred_element_type=jnp.float32)
        t = t + bd_ref[...]

        zero1 = jnp.zeros((1, _G4), jnp.float32)
        zero4 = jnp.zeros((4, _G4), jnp.float32)
        tp1 = jnp.concatenate([t[1:], zero1], axis=0)
        tm1 = jnp.concatenate([zero1, t[:-1]], axis=0)
        tp4 = jnp.concatenate([t[4:], zero4], axis=0)
        tm4 = jnp.concatenate([zero4, t[:-4]], axis=0)
        j = jax.lax.broadcasted_iota(jnp.int32, (_NTOK, 1), 0)
        cpos = j % 4
        rpos = (j % 16) // 4
        s0 = jnp.where(cpos < 3, tp1, 0.0)
        s1 = jnp.where(cpos > 0, tm1, 0.0)
        s2 = jnp.where(rpos < 3, tp4, 0.0)
        s3 = jnp.where(rpos > 0, tm4, 0.0)
        ch = jax.lax.broadcasted_iota(jnp.int32, (_NTOK, _G4), 1)
        s = jnp.where(ch < 24, s0,
                      jnp.where(ch < 48, s1, jnp.where(ch < 72, s2, s3)))

        su = jnp.dot(s.astype(jnp.bfloat16), wuT_ref[...],
                     preferred_element_type=jnp.float32)
        out_ref[b] = x_g + su + bu_ref[...]


def kernel(z, x, w_down, b_down, w_up, b_up):
    B, N_t, C = z.shape
    N_s = x.shape[1]

    n = jax.lax.broadcasted_iota(jnp.int32, (_WIN, _NS), 1)
    w = jax.lax.broadcasted_iota(jnp.int32, (_WIN, _NS), 0)
    pool = (((n // 128) == (w // 8)) & (((n % 32) // 4) == (w % 8))
            ).astype(jnp.bfloat16)
    eye = jnp.eye(_WIN, dtype=jnp.float32)

    out = pl.pallas_call(
        _body,
        grid=(B // _NB,),
        in_specs=[
            pl.BlockSpec((_NB, N_t, C), lambda b: (b, 0, 0)),
            pl.BlockSpec((_NB, N_s, C), lambda b: (b, 0, 0)),
            pl.BlockSpec((_WIN, _NS), lambda b: (0, 0)),
            pl.BlockSpec((_WIN, _WIN), lambda b: (0, 0)),
            pl.BlockSpec((C, _G4), lambda b: (0, 0)),
            pl.BlockSpec((1, _G4), lambda b: (0, 0)),
            pl.BlockSpec((_G4, C), lambda b: (0, 0)),
            pl.BlockSpec((1, C), lambda b: (0, 0)),
        ],
        out_specs=pl.BlockSpec((_NB, _NTOK, C), lambda b: (b, 0, 0)),
        out_shape=jax.ShapeDtypeStruct((B, _NTOK, C), jnp.float32),
        scratch_shapes=[pltpu.VMEM((_NB * _NTOK, C), jnp.float32)],
        compiler_params=pltpu.CompilerParams(
            dimension_semantics=("arbitrary",),
        ),
    )(z, x, pool, eye, w_down.T.astype(jnp.bfloat16), b_down.reshape(1, -1),
      w_up.T.astype(jnp.bfloat16), b_up.reshape(1, -1))
    return out


# speedup vs baseline: 1.5008x; 1.0349x over previous
import jax
import jax.numpy as jnp
from jax.experimental import pallas as pl
from jax.experimental.pallas import tpu as pltpu

_TOPK = 16
_WS = 4
_WIN = 64
_NS = 1024
_C = 384
_G4 = 96
_NTOK = _TOPK * _WS * _WS
_NB = 8


def _body(z_ref, x_ref, pool_ref, eye_ref, wdT_ref, bd_ref, wuT_ref, bu_ref,
          out_ref, xg_ref):

    ranks = []
    for b in range(_NB):
        zmax = jnp.max(z_ref[b], axis=0, keepdims=True)
        zr = zmax.astype(jnp.bfloat16).astype(jnp.float32)
        xb = x_ref[b].astype(jnp.bfloat16)
        A = jnp.dot(pool_ref[...], xb,
                    preferred_element_type=jnp.float32)
        wsum = jnp.sum(A * zr, axis=1, keepdims=True)
        wrow = jnp.transpose(wsum)

        iv = jax.lax.broadcasted_iota(jnp.int32, (_WIN, _WIN), 0)
        iw = jax.lax.broadcasted_iota(jnp.int32, (_WIN, _WIN), 1)
        beats = (wsum > wrow) | ((wsum == wrow) & (iv < iw))
        ranks.append(jnp.sum(beats.astype(jnp.int32), axis=0,
                             keepdims=True))

    iota_row = jax.lax.broadcasted_iota(jnp.int32, (1, _WIN), 1)
    for i in range(_TOPK):
        for b in range(_NB):
            sel = jnp.max(jnp.where(ranks[b] == i, iota_row, 0))
            wh = sel // 8
            ww = sel % 8
            aligned = wh * 128 + (ww // 2) * 8
            odd = (ww % 2) == 1
            for r in range(_WS):
                chunk = x_ref[b, pl.ds(aligned + r * 32, 8), :]
                xg_ref[pl.ds(b * _NTOK + i * 16 + r * 4, 4), :] = jnp.where(
                    odd, chunk[4:8], chunk[0:4])

    for b in range(_NB):
        x_g = xg_ref[pl.ds(b * _NTOK, _NTOK), :]

        t = jnp.dot(x_g.astype(jnp.bfloat16), wdT_ref[...],
                    preferred_element_type=jnp.float32)
        t = t + bd_ref[...]

        zero1 = jnp.zeros((1, _G4), jnp.float32)
        zero4 = jnp.zeros((4, _G4), jnp.float32)
        tp1 = jnp.concatenate([t[1:], zero1], axis=0)
        tm1 = jnp.concatenate([zero1, t[:-1]], axis=0)
        tp4 = jnp.concatenate([t[4:], zero4], axis=0)
        tm4 = jnp.concatenate([zero4, t[:-4]], axis=0)
        j = jax.lax.broadcasted_iota(jnp.int32, (_NTOK, 1), 0)
        cpos = j % 4
        rpos = (j % 16) // 4
        s0 = jnp.where(cpos < 3, tp1, 0.0)
        s1 = jnp.where(cpos > 0, tm1, 0.0)
        s2 = jnp.where(rpos < 3, tp4, 0.0)
        s3 = jnp.where(rpos > 0, tm4, 0.0)
        ch = jax.lax.broadcasted_iota(jnp.int32, (_NTOK, _G4), 1)
        s = jnp.where(ch < 24, s0,
                      jnp.where(ch < 48, s1, jnp.where(ch < 72, s2, s3)))

        su = jnp.dot(s.astype(jnp.bfloat16), wuT_ref[...],
                     preferred_element_type=jnp.float32)
        out_ref[b] = x_g + su + bu_ref[...]


def kernel(z, x, w_down, b_down, w_up, b_up):
    B, N_t, C = z.shape
    N_s = x.shape[1]

    n = jax.lax.broadcasted_iota(jnp.int32, (_WIN, _NS), 1)
    w = jax.lax.broadcasted_iota(jnp.int32, (_WIN, _NS), 0)
    pool = (((n // 128) == (w // 8)) & (((n % 32) // 4) == (w % 8))
            ).astype(jnp.bfloat16)
    eye = jnp.eye(_WIN, dtype=jnp.float32)

    out = pl.pallas_call(
        _body,
        grid=(B // _NB,),
        in_specs=[
            pl.BlockSpec((_NB, N_t, C), lambda b: (b, 0, 0)),
            pl.BlockSpec((_NB, N_s, C), lambda b: (b, 0, 0)),
            pl.BlockSpec((_WIN, _NS), lambda b: (0, 0)),
            pl.BlockSpec((_WIN, _WIN), lambda b: (0, 0)),
            pl.BlockSpec((C, _G4), lambda b: (0, 0)),
            pl.BlockSpec((1, _G4), lambda b: (0, 0)),
            pl.BlockSpec((_G4, C), lambda b: (0, 0)),
            pl.BlockSpec((1, C), lambda b: (0, 0)),
        ],
        out_specs=pl.BlockSpec((_NB, _NTOK, C), lambda b: (b, 0, 0)),
        out_shape=jax.ShapeDtypeStruct((B, _NTOK, C), jnp.float32),
        scratch_shapes=[pltpu.VMEM((_NB * _NTOK, C), jnp.float32)],
        compiler_params=pltpu.CompilerParams(
            dimension_semantics=("arbitrary",),
        ),
    )(z, x, pool, eye, w_down.T.astype(jnp.bfloat16), b_down.reshape(1, -1),
      w_up.T.astype(jnp.bfloat16), b_up.reshape(1, -1))
    return out
